# baseline (speedup 1.0000x reference)
import functools

import jax
import jax.numpy as jnp
from jax import lax
from jax.experimental import pallas as pl
from jax.experimental.pallas import tpu as pltpu
from jax.experimental.pallas import tpu_sc as plsc

NR_HASH = 1000000
EMBED_DIM = 64
HIDDEN = 128
NUM_CLASSES = 10
NB_KEYS = 3
BATCH = 16384
HIST = 50
HISTP = 128

IDX_PER_B = NB_KEYS * HIST
NC, NS = 2, 16
NW = NC * NS
B_PER_W = BATCH // NW
CB = 4
ROWS_PER_CHUNK = CB * IDX_PER_B
DMAS_PER_CHUNK = NB_KEYS * CB
CHUNKS = B_PER_W // CB


def _sc_pool(xp, emb):
    mesh = plsc.VectorSubcoreMesh(core_axis_name="c", subcore_axis_name="s")
    UNROLL = 5
    RITERS = HIST // UNROLL

    def body(x_hbm, emb_hbm, out_hbm, idx_v, rows_v, out_v,
             sem0, sem1, isem0, isem1, osem0, osem1):
        wid = lax.axis_index("s") * NC + lax.axis_index("c")
        base_b = wid * B_PER_W
        sems = (sem0, sem1)
        isems = (isem0, isem1)
        osems = (osem0, osem1)

        z = jnp.zeros((16,), jnp.float32)
        for p in range(2):
            for bb in range(CB):
                for d in range(8):
                    out_v[p, bb, pl.ds(d * 16, 16)] = z

        def idx_copy(c, p):
            b0 = base_b + c * CB
            pltpu.async_copy(x_hbm.at[:, pl.ds(b0, CB)], idx_v.at[p], isems[p])

        def issue(c, p):
            pltpu.make_async_copy(
                x_hbm.at[:, pl.ds(base_b + c * CB, CB)], idx_v.at[p], isems[p]
            ).wait()
            for k in range(NB_KEYS):
                for bb in range(CB):
                    pltpu.async_copy(
                        emb_hbm.at[idx_v.at[p].at[k].at[bb].at[pl.ds(0, HIST)]],
                        rows_v.at[p].at[pl.ds((k * CB + bb) * HIST, HIST)],
                        sems[p],
                    )

        def drain(p):
            for j in range(DMAS_PER_CHUNK):
                pltpu.make_async_copy(
                    emb_hbm.at[idx_v.at[p].at[0].at[0].at[pl.ds(0, HIST)]],
                    rows_v.at[p].at[pl.ds(j * HIST, HIST)],
                    sems[p],
                ).wait()

        def reduce_store(c, p):
            rows = rows_v.at[p]
            @pl.when(c >= 2)
            def _():
                pltpu.make_async_copy(
                    out_v.at[p], out_hbm.at[pl.ds(base_b + c * CB, CB)], osems[p]
                ).wait()

            himask = jnp.full((16,), -65536, jnp.int32)
            for bb in range(CB):
                def rbody(r, accs):
                    accs = list(accs)
                    for k in range(NB_KEYS):
                        for u in range(UNROLL):
                            row = (k * CB + bb) * HIST + r * UNROLL + u
                            for h in range(2):
                                v = plsc.bitcast(
                                    rows[row, pl.ds(h * 32, 32)], jnp.int32
                                )
                                lo = plsc.bitcast(v << 16, jnp.float32)
                                hi = plsc.bitcast(v & himask, jnp.float32)
                                s0 = 2 * h + (u % 2) * 4
                                accs[s0] = accs[s0] + lo
                                accs[s0 + 1] = accs[s0 + 1] + hi
                    return tuple(accs)

                accs = lax.fori_loop(0, RITERS, rbody, (z,) * 8)
                for d in range(4):
                    out_v[p, bb, pl.ds(d * 16, 16)] = accs[d] + accs[d + 4]
            pltpu.async_copy(
                out_v.at[p], out_hbm.at[pl.ds(base_b + c * CB, CB)], osems[p]
            )

        idx_copy(0, 0)
        idx_copy(1, 1)
        issue(0, 0)
        issue(1, 1)

        def outer(g, _):
            for p in range(2):
                c = g * 2 + p
                drain(p)
                @pl.when(c + 2 < CHUNKS)
                def _():
                    idx_copy(c + 2, p)

                reduce_store(c, p)

                @pl.when(c + 2 < CHUNKS)
                def _():
                    issue(c + 2, p)
            return 0

        lax.fori_loop(0, CHUNKS // 2, outer, 0)

        for p in range(2):
            pltpu.make_async_copy(
                out_v.at[p],
                out_hbm.at[pl.ds(base_b + (CHUNKS - 2 + p) * CB, CB)],
                osems[p],
            ).wait()

    return pl.kernel(
        body,
        out_type=jax.ShapeDtypeStruct((BATCH, HISTP), jnp.float32),
        mesh=mesh,
        scratch_types=[
            pltpu.VMEM((2, NB_KEYS, CB, HISTP), jnp.int32),
            pltpu.VMEM((2, ROWS_PER_CHUNK, EMBED_DIM), jnp.bfloat16),
            pltpu.VMEM((2, CB, HISTP), jnp.float32),
            pltpu.SemaphoreType.DMA,
            pltpu.SemaphoreType.DMA,
            pltpu.SemaphoreType.DMA,
            pltpu.SemaphoreType.DMA,
            pltpu.SemaphoreType.DMA,
            pltpu.SemaphoreType.DMA,
        ],
        compiler_params=pltpu.CompilerParams(
            use_tc_tiling_on_sc=False, needs_layout_passes=False
        ),
    )(xp, emb)


TR_BN = 8192
TR_BLOCKS = -(-NR_HASH // TR_BN)
TABLE_ROWS = TR_BLOCKS * TR_BN


def _tr_body(in_ref, o_ref):
    t = jnp.transpose(in_ref[...])
    o_ref[...] = jnp.concatenate(
        [t[: TR_BN // 2], t[TR_BN // 2 :]], axis=-1
    ).astype(jnp.bfloat16)


def _emb_linear(emb):
    embT = emb.T
    out = pl.pallas_call(
        _tr_body,
        grid=(TR_BLOCKS,),
        in_specs=[pl.BlockSpec((EMBED_DIM, TR_BN), lambda i: (0, i))],
        out_specs=pl.BlockSpec((TR_BN // 2, 2 * EMBED_DIM), lambda i: (i, 0)),
        out_shape=jax.ShapeDtypeStruct(
            (TABLE_ROWS // 2, 2 * EMBED_DIM), jnp.bfloat16
        ),
    )(embT)
    return out.reshape(TABLE_ROWS, EMBED_DIM)

def _mlp_body(p_ref, w1_ref, b1_ref, w2_ref, b2_ref, o_ref):
    h = jnp.dot(p_ref[...], w1_ref[...], preferred_element_type=jnp.float32)
    h = jnp.maximum(h + b1_ref[...], 0.0)
    o_ref[...] = (
        jnp.dot(h, w2_ref[...], preferred_element_type=jnp.float32) + b2_ref[...]
    )


def _mlp(pooled, W1p, b1, W2, b2):
    TM = 2048
    grid = (BATCH // TM,)
    return pl.pallas_call(
        _mlp_body,
        grid=grid,
        in_specs=[
            pl.BlockSpec((TM, HISTP), lambda i: (i, 0)),
            pl.BlockSpec((HISTP, HIDDEN), lambda i: (0, 0)),
            pl.BlockSpec((1, HIDDEN), lambda i: (0, 0)),
            pl.BlockSpec((HIDDEN, NUM_CLASSES), lambda i: (0, 0)),
            pl.BlockSpec((1, NUM_CLASSES), lambda i: (0, 0)),
        ],
        out_specs=pl.BlockSpec((TM, NUM_CLASSES), lambda i: (i, 0)),
        out_shape=jax.ShapeDtypeStruct((BATCH, NUM_CLASSES), jnp.float32),
    )(pooled, W1p, b1, W2, b2)


@jax.jit
def kernel(x, emb, W1, b1, W2, b2):
    xq = (x & -8192) + 2 * (x & 4095) + ((x >> 12) & 1)
    xp = jnp.pad(xq, ((0, 0), (0, 0), (0, HISTP - HIST)))
    pooled = _sc_pool(xp, _emb_linear(emb))
    perm = (
        list(range(0, 32, 2)) + list(range(1, 32, 2))
        + list(range(32, 64, 2)) + list(range(33, 64, 2))
    )
    W1p = jnp.zeros((HISTP, HIDDEN), jnp.float32).at[:EMBED_DIM].set(
        (W1 * (1.0 / HIST))[jnp.array(perm)]
    )
    out = _mlp(pooled, W1p, b1.reshape(1, HIDDEN), W2, b2.reshape(1, NUM_CLASSES))
    return out


# speedup vs baseline: 1.5801x; 1.5801x over previous
import functools

import jax
import jax.numpy as jnp
from jax import lax
from jax.experimental import pallas as pl
from jax.experimental.pallas import tpu as pltpu
from jax.experimental.pallas import tpu_sc as plsc

NR_HASH = 1000000
EMBED_DIM = 64
HIDDEN = 128
NUM_CLASSES = 10
NB_KEYS = 3
BATCH = 16384
HIST = 50
HISTP = 128

IDX_PER_B = NB_KEYS * HIST
NC, NS = 2, 16
NW = NC * NS
B_PER_W = BATCH // NW
CB = 4
ROWS_PER_CHUNK = CB * IDX_PER_B
DMAS_PER_CHUNK = NB_KEYS * CB
CHUNKS = B_PER_W // CB


def _sc_pool(xp, emb):
    mesh = plsc.VectorSubcoreMesh(core_axis_name="c", subcore_axis_name="s")
    UNROLL = 5
    RITERS = HIST // UNROLL

    def body(x_hbm, emb_hbm, out_hbm, idx_v, rows_v, out_v,
             sem0, sem1, isem0, isem1, osem0, osem1):
        wid = lax.axis_index("s") * NC + lax.axis_index("c")
        base_b = wid * B_PER_W
        sems = (sem0, sem1)
        isems = (isem0, isem1)
        osems = (osem0, osem1)

        z = jnp.zeros((16,), jnp.float32)
        for p in range(2):
            for bb in range(CB):
                for d in range(8):
                    out_v[p, bb, pl.ds(d * 16, 16)] = z

        def idx_copy(c, p):
            b0 = base_b + c * CB
            pltpu.async_copy(x_hbm.at[:, pl.ds(b0, CB)], idx_v.at[p], isems[p])

        def issue(c, p):
            pltpu.make_async_copy(
                x_hbm.at[:, pl.ds(base_b + c * CB, CB)], idx_v.at[p], isems[p]
            ).wait()
            for k in range(NB_KEYS):
                for bb in range(CB):
                    pltpu.async_copy(
                        emb_hbm.at[idx_v.at[p].at[k].at[bb].at[pl.ds(0, HIST)]],
                        rows_v.at[p].at[pl.ds((k * CB + bb) * HIST, HIST)],
                        sems[p],
                    )

        def drain(p):
            for j in range(DMAS_PER_CHUNK):
                pltpu.make_async_copy(
                    emb_hbm.at[idx_v.at[p].at[0].at[0].at[pl.ds(0, HIST)]],
                    rows_v.at[p].at[pl.ds(j * HIST, HIST)],
                    sems[p],
                ).wait()

        def reduce_store(c, p):
            rows = rows_v.at[p]
            @pl.when(c >= 2)
            def _():
                pltpu.make_async_copy(
                    out_v.at[p], out_hbm.at[pl.ds(base_b + c * CB, CB)], osems[p]
                ).wait()

            for bb in range(CB):
                def rbody(r, accs):
                    accs = list(accs)
                    for k in range(NB_KEYS):
                        for u in range(UNROLL):
                            row = (k * CB + bb) * HIST + r * UNROLL + u
                            for d in range(4):
                                s = d + (u % 2) * 4
                                accs[s] = accs[s] + rows[row, pl.ds(d * 16, 16)]
                    return tuple(accs)

                accs = lax.fori_loop(0, RITERS, rbody, (z,) * 8)
                for d in range(4):
                    out_v[p, bb, pl.ds(d * 16, 16)] = accs[d] + accs[d + 4]
            pltpu.async_copy(
                out_v.at[p], out_hbm.at[pl.ds(base_b + c * CB, CB)], osems[p]
            )

        idx_copy(0, 0)
        idx_copy(1, 1)
        issue(0, 0)
        issue(1, 1)

        def outer(g, _):
            for p in range(2):
                c = g * 2 + p
                drain(p)
                @pl.when(c + 2 < CHUNKS)
                def _():
                    idx_copy(c + 2, p)

                reduce_store(c, p)

                @pl.when(c + 2 < CHUNKS)
                def _():
                    issue(c + 2, p)
            return 0

        lax.fori_loop(0, CHUNKS // 2, outer, 0)

        for p in range(2):
            pltpu.make_async_copy(
                out_v.at[p],
                out_hbm.at[pl.ds(base_b + (CHUNKS - 2 + p) * CB, CB)],
                osems[p],
            ).wait()

    return pl.kernel(
        body,
        out_type=jax.ShapeDtypeStruct((BATCH, HISTP), jnp.float32),
        mesh=mesh,
        scratch_types=[
            pltpu.VMEM((2, NB_KEYS, CB, HISTP), jnp.int32),
            pltpu.VMEM((2, ROWS_PER_CHUNK, EMBED_DIM), jnp.float32),
            pltpu.VMEM((2, CB, HISTP), jnp.float32),
            pltpu.SemaphoreType.DMA,
            pltpu.SemaphoreType.DMA,
            pltpu.SemaphoreType.DMA,
            pltpu.SemaphoreType.DMA,
            pltpu.SemaphoreType.DMA,
            pltpu.SemaphoreType.DMA,
        ],
        compiler_params=pltpu.CompilerParams(use_tc_tiling_on_sc=False),
    )(xp, emb)


TR_BN = 16384
TR_BLOCKS = -(-NR_HASH // TR_BN)
TABLE_ROWS = TR_BLOCKS * TR_BN


def _tr_body(in_ref, o_ref):
    t = jnp.transpose(in_ref[...])
    o_ref[...] = jnp.concatenate([t[: TR_BN // 2], t[TR_BN // 2 :]], axis=-1)


def _emb_linear(emb):
    embT = emb.T
    out = pl.pallas_call(
        _tr_body,
        grid=(TR_BLOCKS,),
        in_specs=[pl.BlockSpec((EMBED_DIM, TR_BN), lambda i: (0, i))],
        out_specs=pl.BlockSpec((TR_BN // 2, 2 * EMBED_DIM), lambda i: (i, 0)),
        out_shape=jax.ShapeDtypeStruct(
            (TABLE_ROWS // 2, 2 * EMBED_DIM), jnp.float32
        ),
    )(embT)
    return out.reshape(TABLE_ROWS, EMBED_DIM)

def _mlp_body(p_ref, w1_ref, b1_ref, w2_ref, b2_ref, o_ref):
    h = jnp.dot(p_ref[...], w1_ref[...], preferred_element_type=jnp.float32)
    h = jnp.maximum(h + b1_ref[...], 0.0)
    o_ref[...] = (
        jnp.dot(h, w2_ref[...], preferred_element_type=jnp.float32) + b2_ref[...]
    )


def _mlp(pooled, W1p, b1, W2, b2):
    TM = 2048
    grid = (BATCH // TM,)
    return pl.pallas_call(
        _mlp_body,
        grid=grid,
        in_specs=[
            pl.BlockSpec((TM, HISTP), lambda i: (i, 0)),
            pl.BlockSpec((HISTP, HIDDEN), lambda i: (0, 0)),
            pl.BlockSpec((1, HIDDEN), lambda i: (0, 0)),
            pl.BlockSpec((HIDDEN, NUM_CLASSES), lambda i: (0, 0)),
            pl.BlockSpec((1, NUM_CLASSES), lambda i: (0, 0)),
        ],
        out_specs=pl.BlockSpec((TM, NUM_CLASSES), lambda i: (i, 0)),
        out_shape=jax.ShapeDtypeStruct((BATCH, NUM_CLASSES), jnp.float32),
    )(pooled, W1p, b1, W2, b2)


@jax.jit
def kernel(x, emb, W1, b1, W2, b2):
    xq = (x & -TR_BN) + 2 * (x & (TR_BN // 2 - 1)) + ((x >> 13) & 1)
    xp = jnp.pad(xq, ((0, 0), (0, 0), (0, HISTP - HIST)))
    pooled = _sc_pool(xp, _emb_linear(emb))
    W1p = jnp.zeros((HISTP, HIDDEN), jnp.float32).at[:EMBED_DIM].set(W1 * (1.0 / HIST))
    out = _mlp(pooled, W1p, b1.reshape(1, HIDDEN), W2, b2.reshape(1, NUM_CLASSES))
    return out


# speedup vs baseline: 1.6305x; 1.0319x over previous
import functools

import jax
import jax.numpy as jnp
from jax import lax
from jax.experimental import pallas as pl
from jax.experimental.pallas import tpu as pltpu
from jax.experimental.pallas import tpu_sc as plsc

NR_HASH = 1000000
EMBED_DIM = 64
HIDDEN = 128
NUM_CLASSES = 10
NB_KEYS = 3
BATCH = 16384
HIST = 50
HISTP = 128

IDX_PER_B = NB_KEYS * HIST
NC, NS = 2, 16
NW = NC * NS
B_PER_W = BATCH // NW
CB = 4
ROWS_PER_CHUNK = CB * IDX_PER_B
DMAS_PER_CHUNK = NB_KEYS * CB
CHUNKS = B_PER_W // CB


def _sc_pool(xp, emb):
    mesh = plsc.VectorSubcoreMesh(core_axis_name="c", subcore_axis_name="s")
    UNROLL = 5
    RITERS = HIST // UNROLL

    def body(x_hbm, emb_hbm, out_hbm, idx_v, rows_v, out_v,
             sem0, sem1, isem0, isem1, osem0, osem1):
        wid = lax.axis_index("s") * NC + lax.axis_index("c")
        base_b = wid * B_PER_W
        sems = (sem0, sem1)
        isems = (isem0, isem1)
        osems = (osem0, osem1)

        z = jnp.zeros((16,), jnp.float32)
        for p in range(2):
            for bb in range(CB):
                for d in range(8):
                    out_v[p, bb, pl.ds(d * 16, 16)] = z

        def idx_copy(c, p):
            b0 = base_b + c * CB
            pltpu.async_copy(x_hbm.at[:, pl.ds(b0, CB)], idx_v.at[p], isems[p])

        def issue(c, p):
            pltpu.make_async_copy(
                x_hbm.at[:, pl.ds(base_b + c * CB, CB)], idx_v.at[p], isems[p]
            ).wait()
            for k in range(NB_KEYS):
                for bb in range(CB):
                    pltpu.async_copy(
                        emb_hbm.at[idx_v.at[p].at[k].at[bb].at[pl.ds(0, HIST)]],
                        rows_v.at[p].at[pl.ds((k * CB + bb) * HIST, HIST)],
                        sems[p],
                    )

        def drain(p):
            for j in range(DMAS_PER_CHUNK):
                pltpu.make_async_copy(
                    emb_hbm.at[idx_v.at[p].at[0].at[0].at[pl.ds(0, HIST)]],
                    rows_v.at[p].at[pl.ds(j * HIST, HIST)],
                    sems[p],
                ).wait()

        def reduce_store(c, p):
            rows = rows_v.at[p]
            @pl.when(c >= 2)
            def _():
                pltpu.make_async_copy(
                    out_v.at[p], out_hbm.at[pl.ds(base_b + c * CB, CB)], osems[p]
                ).wait()

            for bb in range(CB):
                def rbody(r, accs):
                    accs = list(accs)
                    for k in range(NB_KEYS):
                        for u in range(UNROLL):
                            row = (k * CB + bb) * HIST + r * UNROLL + u
                            for d in range(4):
                                s = d + (u % 2) * 4
                                accs[s] = accs[s] + rows[row, pl.ds(d * 16, 16)]
                    return tuple(accs)

                accs = lax.fori_loop(0, RITERS, rbody, (z,) * 8)
                for d in range(4):
                    out_v[p, bb, pl.ds(d * 16, 16)] = accs[d] + accs[d + 4]
            pltpu.async_copy(
                out_v.at[p], out_hbm.at[pl.ds(base_b + c * CB, CB)], osems[p]
            )

        idx_copy(0, 0)
        idx_copy(1, 1)
        issue(0, 0)
        issue(1, 1)

        def outer(g, _):
            for p in range(2):
                c = g * 2 + p
                drain(p)
                @pl.when(c + 2 < CHUNKS)
                def _():
                    idx_copy(c + 2, p)

                reduce_store(c, p)

                @pl.when(c + 2 < CHUNKS)
                def _():
                    issue(c + 2, p)
            return 0

        lax.fori_loop(0, CHUNKS // 2, outer, 0)

        for p in range(2):
            pltpu.make_async_copy(
                out_v.at[p],
                out_hbm.at[pl.ds(base_b + (CHUNKS - 2 + p) * CB, CB)],
                osems[p],
            ).wait()

    return pl.kernel(
        body,
        out_type=jax.ShapeDtypeStruct((BATCH, HISTP), jnp.float32),
        mesh=mesh,
        scratch_types=[
            pltpu.VMEM((2, NB_KEYS, CB, HISTP), jnp.int32),
            pltpu.VMEM((2, ROWS_PER_CHUNK, EMBED_DIM), jnp.float32),
            pltpu.VMEM((2, CB, HISTP), jnp.float32),
            pltpu.SemaphoreType.DMA,
            pltpu.SemaphoreType.DMA,
            pltpu.SemaphoreType.DMA,
            pltpu.SemaphoreType.DMA,
            pltpu.SemaphoreType.DMA,
            pltpu.SemaphoreType.DMA,
        ],
        compiler_params=pltpu.CompilerParams(use_tc_tiling_on_sc=False),
    )(xp, emb)


TR_BN = 32768
TR_BLOCKS = -(-NR_HASH // TR_BN)
TABLE_ROWS = TR_BLOCKS * TR_BN


def _tr_body(in_ref, o_ref):
    t = jnp.transpose(in_ref[...])
    o_ref[...] = jnp.concatenate([t[: TR_BN // 2], t[TR_BN // 2 :]], axis=-1)


def _emb_linear(emb):
    embT = emb.T
    out = pl.pallas_call(
        _tr_body,
        grid=(TR_BLOCKS,),
        in_specs=[pl.BlockSpec((EMBED_DIM, TR_BN), lambda i: (0, i))],
        out_specs=pl.BlockSpec((TR_BN // 2, 2 * EMBED_DIM), lambda i: (i, 0)),
        out_shape=jax.ShapeDtypeStruct(
            (TABLE_ROWS // 2, 2 * EMBED_DIM), jnp.float32
        ),
    )(embT)
    return out.reshape(TABLE_ROWS, EMBED_DIM)

def _mlp_body(p_ref, w1_ref, b1_ref, w2_ref, b2_ref, o_ref):
    h = jnp.dot(p_ref[...], w1_ref[...], preferred_element_type=jnp.float32)
    h = jnp.maximum(h + b1_ref[...], 0.0)
    o_ref[...] = (
        jnp.dot(h, w2_ref[...], preferred_element_type=jnp.float32) + b2_ref[...]
    )


def _mlp(pooled, W1p, b1, W2, b2):
    TM = 2048
    grid = (BATCH // TM,)
    return pl.pallas_call(
        _mlp_body,
        grid=grid,
        in_specs=[
            pl.BlockSpec((TM, HISTP), lambda i: (i, 0)),
            pl.BlockSpec((HISTP, HIDDEN), lambda i: (0, 0)),
            pl.BlockSpec((1, HIDDEN), lambda i: (0, 0)),
            pl.BlockSpec((HIDDEN, NUM_CLASSES), lambda i: (0, 0)),
            pl.BlockSpec((1, NUM_CLASSES), lambda i: (0, 0)),
        ],
        out_specs=pl.BlockSpec((TM, NUM_CLASSES), lambda i: (i, 0)),
        out_shape=jax.ShapeDtypeStruct((BATCH, NUM_CLASSES), jnp.float32),
    )(pooled, W1p, b1, W2, b2)


@jax.jit
def kernel(x, emb, W1, b1, W2, b2):
    xq = (x & -TR_BN) + 2 * (x & (TR_BN // 2 - 1)) + ((x >> 14) & 1)
    xp = jnp.pad(xq, ((0, 0), (0, 0), (0, HISTP - HIST)))
    pooled = _sc_pool(xp, _emb_linear(emb))
    W1p = jnp.zeros((HISTP, HIDDEN), jnp.float32).at[:EMBED_DIM].set(W1 * (1.0 / HIST))
    out = _mlp(pooled, W1p, b1.reshape(1, HIDDEN), W2, b2.reshape(1, NUM_CLASSES))
    return out


# speedup vs baseline: 1.6316x; 1.0007x over previous
import jax
import jax.numpy as jnp
from jax import lax
from jax.experimental import pallas as pl
from jax.experimental.pallas import tpu as pltpu
from jax.experimental.pallas import tpu_sc as plsc

NR_HASH = 1000000
EMBED_DIM = 64
HIDDEN = 128
NUM_CLASSES = 10
NB_KEYS = 3
BATCH = 16384
HIST = 50
HISTP = 128

IDX_PER_B = NB_KEYS * HIST
NC, NS = 2, 16
NW = NC * NS
B_PER_W = BATCH // NW
CB = 4
ROWS_PER_CHUNK = CB * IDX_PER_B
DMAS_PER_CHUNK = NB_KEYS * CB
CHUNKS = B_PER_W // CB


def _sc_pool(xp, emb):
    mesh = plsc.VectorSubcoreMesh(core_axis_name="c", subcore_axis_name="s")
    UNROLL = 5
    RITERS = HIST // UNROLL

    def body(x_hbm, emb_hbm, out_hbm, idx_v, rows_v, out_v,
             sem0, sem1, isem0, isem1, osem0, osem1):
        wid = lax.axis_index("s") * NC + lax.axis_index("c")
        base_b = wid * B_PER_W
        sems = (sem0, sem1)
        isems = (isem0, isem1)
        osems = (osem0, osem1)

        z = jnp.zeros((16,), jnp.float32)
        for p in range(2):
            for bb in range(CB):
                for d in range(8):
                    out_v[p, bb, pl.ds(d * 16, 16)] = z

        def idx_copy(c, p):
            b0 = base_b + c * CB
            pltpu.async_copy(x_hbm.at[:, pl.ds(b0, CB)], idx_v.at[p], isems[p])

        def issue(c, p):
            pltpu.make_async_copy(
                x_hbm.at[:, pl.ds(base_b + c * CB, CB)], idx_v.at[p], isems[p]
            ).wait()
            for k in range(NB_KEYS):
                for bb in range(CB):
                    pltpu.async_copy(
                        emb_hbm.at[idx_v.at[p].at[k].at[bb].at[pl.ds(0, HIST)]],
                        rows_v.at[p].at[pl.ds((k * CB + bb) * HIST, HIST)],
                        sems[p],
                    )

        def drain(p):
            for j in range(DMAS_PER_CHUNK):
                pltpu.make_async_copy(
                    emb_hbm.at[idx_v.at[p].at[0].at[0].at[pl.ds(0, HIST)]],
                    rows_v.at[p].at[pl.ds(j * HIST, HIST)],
                    sems[p],
                ).wait()

        def reduce_store(c, p):
            rows = rows_v.at[p]
            @pl.when(c >= 2)
            def _():
                pltpu.make_async_copy(
                    out_v.at[p], out_hbm.at[pl.ds(base_b + c * CB, CB)], osems[p]
                ).wait()

            for bb in range(CB):
                def rbody(r, accs):
                    accs = list(accs)
                    for k in range(NB_KEYS):
                        for u in range(UNROLL):
                            row = (k * CB + bb) * HIST + r * UNROLL + u
                            for d in range(4):
                                s = d + (u % 2) * 4
                                accs[s] = accs[s] + rows[row, pl.ds(d * 16, 16)]
                    return tuple(accs)

                accs = lax.fori_loop(0, RITERS, rbody, (z,) * 8)
                for d in range(4):
                    out_v[p, bb, pl.ds(d * 16, 16)] = accs[d] + accs[d + 4]
            pltpu.async_copy(
                out_v.at[p], out_hbm.at[pl.ds(base_b + c * CB, CB)], osems[p]
            )

        idx_copy(0, 0)
        idx_copy(1, 1)
        issue(0, 0)
        issue(1, 1)

        def outer(g, _):
            for p in range(2):
                c = g * 2 + p
                drain(p)
                @pl.when(c + 2 < CHUNKS)
                def _():
                    idx_copy(c + 2, p)

                reduce_store(c, p)

                @pl.when(c + 2 < CHUNKS)
                def _():
                    issue(c + 2, p)
            return 0

        lax.fori_loop(0, CHUNKS // 2, outer, 0)

        for p in range(2):
            pltpu.make_async_copy(
                out_v.at[p],
                out_hbm.at[pl.ds(base_b + (CHUNKS - 2 + p) * CB, CB)],
                osems[p],
            ).wait()

    return pl.kernel(
        body,
        out_type=jax.ShapeDtypeStruct((BATCH, HISTP), jnp.float32),
        mesh=mesh,
        scratch_types=[
            pltpu.VMEM((2, NB_KEYS, CB, HISTP), jnp.int32),
            pltpu.VMEM((2, ROWS_PER_CHUNK, EMBED_DIM), jnp.float32),
            pltpu.VMEM((2, CB, HISTP), jnp.float32),
            pltpu.SemaphoreType.DMA,
            pltpu.SemaphoreType.DMA,
            pltpu.SemaphoreType.DMA,
            pltpu.SemaphoreType.DMA,
            pltpu.SemaphoreType.DMA,
            pltpu.SemaphoreType.DMA,
        ],
        compiler_params=pltpu.CompilerParams(use_tc_tiling_on_sc=False),
    )(xp, emb)


TR_BN = 32768
TR_BLOCKS = -(-NR_HASH // TR_BN)
TABLE_ROWS = TR_BLOCKS * TR_BN


def _tr_body(in_ref, o_ref):
    t = jnp.transpose(in_ref[...])
    o_ref[...] = jnp.concatenate([t[: TR_BN // 2], t[TR_BN // 2 :]], axis=-1)


def _emb_linear(emb):
    embT = emb.T
    out = pl.pallas_call(
        _tr_body,
        grid=(TR_BLOCKS,),
        in_specs=[pl.BlockSpec((EMBED_DIM, TR_BN), lambda i: (0, i))],
        out_specs=pl.BlockSpec((TR_BN // 2, 2 * EMBED_DIM), lambda i: (i, 0)),
        out_shape=jax.ShapeDtypeStruct(
            (TABLE_ROWS // 2, 2 * EMBED_DIM), jnp.float32
        ),
    )(embT)
    return out.reshape(TABLE_ROWS, EMBED_DIM)

def _mlp_body(p_ref, w1_ref, b1_ref, w2_ref, b2_ref, o_ref):
    h = jnp.dot(p_ref[...], w1_ref[...], preferred_element_type=jnp.float32)
    h = jnp.maximum(h + b1_ref[...], 0.0)
    o_ref[...] = (
        jnp.dot(h, w2_ref[...], preferred_element_type=jnp.float32) + b2_ref[...]
    )


def _mlp(pooled, W1p, b1, W2, b2):
    TM = 2048
    grid = (BATCH // TM,)
    return pl.pallas_call(
        _mlp_body,
        grid=grid,
        in_specs=[
            pl.BlockSpec((TM, HISTP), lambda i: (i, 0)),
            pl.BlockSpec((HISTP, HIDDEN), lambda i: (0, 0)),
            pl.BlockSpec((1, HIDDEN), lambda i: (0, 0)),
            pl.BlockSpec((HIDDEN, NUM_CLASSES), lambda i: (0, 0)),
            pl.BlockSpec((1, NUM_CLASSES), lambda i: (0, 0)),
        ],
        out_specs=pl.BlockSpec((TM, NUM_CLASSES), lambda i: (i, 0)),
        out_shape=jax.ShapeDtypeStruct((BATCH, NUM_CLASSES), jnp.float32),
    )(pooled, W1p, b1, W2, b2)


@jax.jit
def kernel(x, emb, W1, b1, W2, b2):
    xq = (x & -TR_BN) + 2 * (x & (TR_BN // 2 - 1)) + ((x >> 14) & 1)
    xp = jnp.pad(xq, ((0, 0), (0, 0), (0, HISTP - HIST)))
    pooled = _sc_pool(xp, _emb_linear(emb))
    W1p = jnp.zeros((HISTP, HIDDEN), jnp.float32).at[:EMBED_DIM].set(W1 * (1.0 / HIST))
    out = _mlp(pooled, W1p, b1.reshape(1, HIDDEN), W2, b2.reshape(1, NUM_CLASSES))
    return out
